# baseline (speedup 1.0000x reference)
import functools

import jax
import jax.numpy as jnp
from jax import lax
from jax.experimental import pallas as pl
from jax.experimental.pallas import tpu as pltpu
from jax.experimental.pallas import tpu_sc as plsc

N = 10000
E = 320000
D = 128
O = 128

NC = 2
NS = 16
NW = NC * NS
EPW = E // NW
CH = 125
NCHUNK = EPW // CH
PH = 4
CPP = NCHUNK // PH
ROWS_PER_TILE = 632
NPAD = ROWS_PER_TILE * NS

assert EPW * NW == E and CPP * PH * CH == EPW and CPP % 2 == 0 and NPAD >= N


def _build_sc_aggregate():
  mesh = plsc.VectorSubcoreMesh(core_axis_name="c", subcore_axis_name="s")

  @functools.partial(
      pl.kernel,
      out_type=jax.ShapeDtypeStruct((NC, NPAD, D), jnp.float32),
      mesh=mesh,
      scratch_types=[
          pltpu.VMEM((CPP, CH), jnp.int32),
          pltpu.VMEM((CPP, CH), jnp.int32),
          pltpu.VMEM((CH, D), jnp.float32),
          pltpu.VMEM((CH, D), jnp.float32),
          pltpu.VMEM_SHARED((NPAD, D), jnp.float32),
          pltpu.SemaphoreType.DMA,
          pltpu.SemaphoreType.DMA,
          pltpu.SemaphoreType.DMA,
          pltpu.SemaphoreType.DMA,
      ],
  )
  def agg_kernel(nodes_hbm, snd_hbm, rcv_hbm, out_hbm,
                 idx_s, idx_r, rows0, rows1, acc, g0, g1, s0, s1):
    c = lax.axis_index("c")
    s = lax.axis_index("s")
    wid = c * NS + s
    row0 = pl.multiple_of(s * ROWS_PER_TILE, 8)

    zvec = jnp.zeros((16,), jnp.float32)

    @pl.loop(0, CH)
    def _(r):
      for cc in range(D // 16):
        rows0[r, pl.ds(cc * 16, 16)] = zvec

    for k in range(6):
      pltpu.sync_copy(rows0.at[pl.ds(0, 96)], acc.at[pl.ds(row0 + k * 96, 96)])
    pltpu.sync_copy(rows0.at[pl.ds(0, 56)], acc.at[pl.ds(row0 + 576, 56)])
    plsc.subcore_barrier()

    for p in range(PH):
      pltpu.sync_copy(snd_hbm.at[wid, p], idx_s)
      pltpu.sync_copy(rcv_hbm.at[wid, p], idx_r)
      pltpu.async_copy(nodes_hbm.at[idx_s.at[0]], rows0, g0)
      pltpu.async_copy(nodes_hbm.at[idx_s.at[1]], rows1, g1)

      @pl.loop(0, CPP // 2)
      def _(jj):
        j = jj * 2
        pltpu.make_async_copy(nodes_hbm.at[idx_s.at[j]], rows0, g0).wait()
        pltpu.async_copy(rows0, acc.at[idx_r.at[j]], s0, add=True)
        pltpu.make_async_copy(nodes_hbm.at[idx_s.at[j + 1]], rows1, g1).wait()
        pltpu.async_copy(rows1, acc.at[idx_r.at[j + 1]], s1, add=True)

        @pl.when(jj + 1 < CPP // 2)
        def _():
          pltpu.make_async_copy(rows0, acc.at[idx_r.at[j]], s0).wait()
          pltpu.async_copy(nodes_hbm.at[idx_s.at[j + 2]], rows0, g0)
          pltpu.make_async_copy(rows1, acc.at[idx_r.at[j + 1]], s1).wait()
          pltpu.async_copy(nodes_hbm.at[idx_s.at[j + 3]], rows1, g1)

      pltpu.make_async_copy(rows0, acc.at[idx_r.at[CPP - 2]], s0).wait()
      pltpu.make_async_copy(rows1, acc.at[idx_r.at[CPP - 1]], s1).wait()

    plsc.subcore_barrier()
    pltpu.sync_copy(acc.at[pl.ds(row0, ROWS_PER_TILE)],
                    out_hbm.at[c, pl.ds(row0, ROWS_PER_TILE)])

  return agg_kernel


_SC_AGGREGATE = _build_sc_aggregate()

BLK = 2000


def _tc_root_body(x_ref, wr_ref, b_ref, o_ref):
  o_ref[...] = (
      jnp.dot(x_ref[...], wr_ref[...], preferred_element_type=jnp.float32)
      + b_ref[...])


_tc_root = pl.pallas_call(
    _tc_root_body,
    grid=(N // BLK,),
    in_specs=[
        pl.BlockSpec((BLK, D), lambda i: (i, 0)),
        pl.BlockSpec((D, O), lambda i: (0, 0)),
        pl.BlockSpec((1, O), lambda i: (0, 0)),
    ],
    out_specs=pl.BlockSpec((BLK, O), lambda i: (i, 0)),
    out_shape=jax.ShapeDtypeStruct((N, O), jnp.float32),
)


def _tc_combine_body(p_ref, r_ref, w_ref, o_ref):
  aggv = p_ref[0] + p_ref[1]
  o_ref[...] = (
      jnp.dot(aggv, w_ref[...], preferred_element_type=jnp.float32)
      + r_ref[...])


_tc_combine = pl.pallas_call(
    _tc_combine_body,
    grid=(N // BLK,),
    in_specs=[
        pl.BlockSpec((NC, BLK, D), lambda i: (0, i, 0)),
        pl.BlockSpec((BLK, O), lambda i: (i, 0)),
        pl.BlockSpec((D, O), lambda i: (0, 0)),
    ],
    out_specs=pl.BlockSpec((BLK, O), lambda i: (i, 0)),
    out_shape=jax.ShapeDtypeStruct((N, O), jnp.float32),
)


def kernel(nodes, senders, receivers, W, b, W_root):
  snd = senders.reshape(NW, PH, CPP, CH)
  rcv = receivers.reshape(NW, PH, CPP, CH)
  root = _tc_root(nodes, W_root, b.reshape(1, O))
  partials = _SC_AGGREGATE(nodes, snd, rcv)
  return _tc_combine(partials, root, W)


# speedup vs baseline: 1.2363x; 1.2363x over previous
import functools

import jax
import jax.numpy as jnp
from jax import lax
from jax.experimental import pallas as pl
from jax.experimental.pallas import tpu as pltpu
from jax.experimental.pallas import tpu_sc as plsc

N = 10000
E = 320000
D = 128
O = 128

NC = 2
NS = 16
NW = NC * NS
EPW = E // NW
CH = 100
NCHUNK = EPW // CH
PH = 2
CPP = NCHUNK // PH
ROWS_PER_TILE = 632
NPAD = ROWS_PER_TILE * NS

assert EPW * NW == E and CPP * PH * CH == EPW and CPP % 2 == 0 and NPAD >= N


def _build_sc_aggregate():
  mesh = plsc.VectorSubcoreMesh(core_axis_name="c", subcore_axis_name="s")

  @functools.partial(
      pl.kernel,
      out_type=jax.ShapeDtypeStruct((NC, NPAD, D), jnp.float32),
      mesh=mesh,
      scratch_types=[
          pltpu.VMEM((CPP, CH), jnp.int32),
          pltpu.VMEM((CPP, CH), jnp.int32),
          pltpu.VMEM((CH, D), jnp.float32),
          pltpu.VMEM((CH, D), jnp.float32),
          pltpu.VMEM_SHARED((NPAD, D), jnp.float32),
          pltpu.SemaphoreType.DMA,
          pltpu.SemaphoreType.DMA,
      ],
  )
  def agg_kernel(nodes_hbm, snd_hbm, rcv_hbm, out_hbm,
                 idx_s, idx_r, rows0, rows1, acc, g0, g1):
    c = lax.axis_index("c")
    s = lax.axis_index("s")
    wid = c * NS + s
    row0 = pl.multiple_of(s * ROWS_PER_TILE, 8)

    zvec = jnp.zeros((16,), jnp.float32)

    @pl.loop(0, CH)
    def _(r):
      for cc in range(D // 16):
        rows0[r, pl.ds(cc * 16, 16)] = zvec

    for k in range(6):
      pltpu.sync_copy(rows0.at[pl.ds(0, 96)], acc.at[pl.ds(row0 + k * 96, 96)])
    pltpu.sync_copy(rows0.at[pl.ds(0, 56)], acc.at[pl.ds(row0 + 576, 56)])
    plsc.subcore_barrier()

    for p in range(PH):
      pltpu.sync_copy(snd_hbm.at[wid, p], idx_s)
      pltpu.sync_copy(rcv_hbm.at[wid, p], idx_r)
      pltpu.async_copy(nodes_hbm.at[idx_s.at[0]], rows0, g0)

      @pl.loop(0, CPP // 2)
      def _(jj):
        j = jj * 2
        pltpu.async_copy(nodes_hbm.at[idx_s.at[j + 1]], rows1, g1)
        pltpu.make_async_copy(nodes_hbm.at[idx_s.at[j]], rows0, g0).wait()
        pltpu.sync_copy(rows0, acc.at[idx_r.at[j]], add=True)

        @pl.when(jj + 1 < CPP // 2)
        def _():
          pltpu.async_copy(nodes_hbm.at[idx_s.at[j + 2]], rows0, g0)

        pltpu.make_async_copy(nodes_hbm.at[idx_s.at[j + 1]], rows1, g1).wait()
        pltpu.sync_copy(rows1, acc.at[idx_r.at[j + 1]], add=True)

    plsc.subcore_barrier()
    pltpu.sync_copy(acc.at[pl.ds(row0, ROWS_PER_TILE)],
                    out_hbm.at[c, pl.ds(row0, ROWS_PER_TILE)])

  return agg_kernel


_SC_AGGREGATE = _build_sc_aggregate()

BLK = 2000


def _tc_root_body(x_ref, wr_ref, b_ref, o_ref):
  o_ref[...] = (
      jnp.dot(x_ref[...], wr_ref[...], preferred_element_type=jnp.float32)
      + b_ref[...])


_tc_root = pl.pallas_call(
    _tc_root_body,
    grid=(N // BLK,),
    in_specs=[
        pl.BlockSpec((BLK, D), lambda i: (i, 0)),
        pl.BlockSpec((D, O), lambda i: (0, 0)),
        pl.BlockSpec((1, O), lambda i: (0, 0)),
    ],
    out_specs=pl.BlockSpec((BLK, O), lambda i: (i, 0)),
    out_shape=jax.ShapeDtypeStruct((N, O), jnp.float32),
)


def _tc_combine_body(p_ref, r_ref, w_ref, o_ref):
  aggv = p_ref[0] + p_ref[1]
  o_ref[...] = (
      jnp.dot(aggv, w_ref[...], preferred_element_type=jnp.float32)
      + r_ref[...])


_tc_combine = pl.pallas_call(
    _tc_combine_body,
    grid=(N // BLK,),
    in_specs=[
        pl.BlockSpec((NC, BLK, D), lambda i: (0, i, 0)),
        pl.BlockSpec((BLK, O), lambda i: (i, 0)),
        pl.BlockSpec((D, O), lambda i: (0, 0)),
    ],
    out_specs=pl.BlockSpec((BLK, O), lambda i: (i, 0)),
    out_shape=jax.ShapeDtypeStruct((N, O), jnp.float32),
)


def kernel(nodes, senders, receivers, W, b, W_root):
  snd = senders.reshape(NW, PH, CPP, CH)
  rcv = receivers.reshape(NW, PH, CPP, CH)
  root = _tc_root(nodes, W_root, b.reshape(1, O))
  partials = _SC_AGGREGATE(nodes, snd, rcv)
  return _tc_combine(partials, root, W)


# speedup vs baseline: 1.2723x; 1.0291x over previous
import functools

import jax
import jax.numpy as jnp
from jax import lax
from jax.experimental import pallas as pl
from jax.experimental.pallas import tpu as pltpu
from jax.experimental.pallas import tpu_sc as plsc

N = 10000
E = 320000
D = 128
O = 128

NC = 2
NS = 16
NW = NC * NS
EPW = E // NW
CH = 125
NCHUNK = EPW // CH
PH = 2
CPP = NCHUNK // PH
ROWS_PER_TILE = 632
NPAD = ROWS_PER_TILE * NS

assert EPW * NW == E and CPP * PH * CH == EPW and CPP % 2 == 0 and NPAD >= N


def _build_sc_aggregate():
  mesh = plsc.VectorSubcoreMesh(core_axis_name="c", subcore_axis_name="s")

  @functools.partial(
      pl.kernel,
      out_type=jax.ShapeDtypeStruct((NC, NPAD, D), jnp.float32),
      mesh=mesh,
      scratch_types=[
          pltpu.VMEM((CPP, CH), jnp.int32),
          pltpu.VMEM((CPP, CH), jnp.int32),
          pltpu.VMEM((CH, D), jnp.float32),
          pltpu.VMEM((CH, D), jnp.float32),
          pltpu.VMEM_SHARED((NPAD, D), jnp.float32),
          pltpu.SemaphoreType.DMA,
          pltpu.SemaphoreType.DMA,
      ],
  )
  def agg_kernel(nodes_hbm, snd_hbm, rcv_hbm, out_hbm,
                 idx_s, idx_r, rows0, rows1, acc, g0, g1):
    c = lax.axis_index("c")
    s = lax.axis_index("s")
    wid = c * NS + s
    row0 = pl.multiple_of(s * ROWS_PER_TILE, 8)

    zvec = jnp.zeros((16,), jnp.float32)

    @pl.loop(0, 96)
    def _(r):
      for cc in range(D // 16):
        rows0[r, pl.ds(cc * 16, 16)] = zvec

    for k in range(6):
      pltpu.sync_copy(rows0.at[pl.ds(0, 96)], acc.at[pl.ds(row0 + k * 96, 96)])
    pltpu.sync_copy(rows0.at[pl.ds(0, 56)], acc.at[pl.ds(row0 + 576, 56)])
    plsc.subcore_barrier()

    for p in range(PH):
      pltpu.sync_copy(snd_hbm.at[wid, p], idx_s)
      pltpu.sync_copy(rcv_hbm.at[wid, p], idx_r)
      pltpu.async_copy(nodes_hbm.at[idx_s.at[0]], rows0, g0)

      @pl.loop(0, CPP // 2)
      def _(jj):
        j = jj * 2
        pltpu.async_copy(nodes_hbm.at[idx_s.at[j + 1]], rows1, g1)
        pltpu.make_async_copy(nodes_hbm.at[idx_s.at[j]], rows0, g0).wait()
        pltpu.sync_copy(rows0, acc.at[idx_r.at[j]], add=True)

        @pl.when(jj + 1 < CPP // 2)
        def _():
          pltpu.async_copy(nodes_hbm.at[idx_s.at[j + 2]], rows0, g0)

        pltpu.make_async_copy(nodes_hbm.at[idx_s.at[j + 1]], rows1, g1).wait()
        pltpu.sync_copy(rows1, acc.at[idx_r.at[j + 1]], add=True)

    plsc.subcore_barrier()
    pltpu.sync_copy(acc.at[pl.ds(row0, ROWS_PER_TILE)],
                    out_hbm.at[c, pl.ds(row0, ROWS_PER_TILE)])

  return agg_kernel


_SC_AGGREGATE = _build_sc_aggregate()

BLK = 2000


def _tc_combine_body(p_ref, x_ref, w_ref, wr_ref, b_ref, o_ref):
  aggv = p_ref[0] + p_ref[1]
  o_ref[...] = (
      jnp.dot(aggv, w_ref[...], preferred_element_type=jnp.float32)
      + jnp.dot(x_ref[...], wr_ref[...], preferred_element_type=jnp.float32)
      + b_ref[...])


_tc_combine = pl.pallas_call(
    _tc_combine_body,
    grid=(N // BLK,),
    in_specs=[
        pl.BlockSpec((NC, BLK, D), lambda i: (0, i, 0)),
        pl.BlockSpec((BLK, D), lambda i: (i, 0)),
        pl.BlockSpec((D, O), lambda i: (0, 0)),
        pl.BlockSpec((D, O), lambda i: (0, 0)),
        pl.BlockSpec((1, O), lambda i: (0, 0)),
    ],
    out_specs=pl.BlockSpec((BLK, O), lambda i: (i, 0)),
    out_shape=jax.ShapeDtypeStruct((N, O), jnp.float32),
)


def kernel(nodes, senders, receivers, W, b, W_root):
  snd = senders.reshape(NW, PH, CPP, CH)
  rcv = receivers.reshape(NW, PH, CPP, CH)
  partials = _SC_AGGREGATE(nodes, snd, rcv)
  return _tc_combine(partials, nodes, W, W_root, b.reshape(1, O))
